# baseline (speedup 1.0000x reference)
import functools

import jax
import jax.numpy as jnp
from jax import lax
from jax.experimental import pallas as pl
from jax.experimental.pallas import tpu as pltpu
from jax.experimental.pallas import tpu_sc as plsc

NUM_ROWS = 16384
EMB = 64
NUM_COLS = 5 * EMB
PAD = 128
MASK_ID = 1
NUM_CORES = 2
NUM_SUBCORES = 16
NW = NUM_CORES * NUM_SUBCORES
ROWS_PER_W = NUM_ROWS // NW
CH = 128
NCH = ROWS_PER_W // CH


_TB = 8192


def _tc_relayout(table):
    v = table.shape[0]
    t_t = table.T

    def body(t_ref, out_ref):
        x = t_ref[...]
        xt = jnp.swapaxes(x, 0, 1)
        out_ref[...] = jnp.concatenate(
            [xt, jnp.zeros((_TB, EMB), jnp.float32)], axis=1)

    return pl.pallas_call(
        body,
        grid=(pl.cdiv(v, _TB),),
        in_specs=[pl.BlockSpec((EMB, _TB), lambda i: (0, i))],
        out_specs=pl.BlockSpec((_TB, PAD), lambda i: (i, 0)),
        out_shape=jax.ShapeDtypeStruct((v, PAD), jnp.float32),
    )(t_t)


_AB = 2048


def _tc_assemble(g_n, g_h, g_r, g_t):
    def body(n_ref, h_ref, r_ref, t_ref, out_ref):
        n_t = jnp.swapaxes(n_ref[:, :EMB], 0, 1)
        h_t = jnp.swapaxes(h_ref[:, :EMB], 0, 1)
        r_t = jnp.swapaxes(r_ref[:, :EMB], 0, 1)
        t_t = jnp.swapaxes(t_ref[:, :EMB], 0, 1)
        out_ref[...] = jnp.concatenate([n_t, h_t, r_t, n_t, t_t], axis=0)

    spec = pl.BlockSpec((_AB, PAD), lambda j: (j, 0))
    out_t = pl.pallas_call(
        body,
        grid=(NUM_ROWS // _AB,),
        in_specs=[spec, spec, spec, spec],
        out_specs=pl.BlockSpec((NUM_COLS, _AB), lambda j: (0, j)),
        out_shape=jax.ShapeDtypeStruct((NUM_COLS, NUM_ROWS), jnp.float32),
    )(g_n, g_h, g_r, g_t)
    return out_t.T


def _sc_gather2(idx0, idx1, tab0, tab1):
    mesh = plsc.VectorSubcoreMesh(core_axis_name="c", subcore_axis_name="s")
    out_t = jax.ShapeDtypeStruct((NUM_ROWS, PAD), jnp.float32)

    @functools.partial(
        pl.kernel,
        mesh=mesh,
        compiler_params=pltpu.CompilerParams(use_tc_tiling_on_sc=False),
        out_type=(out_t, out_t),
        scratch_types=[
            pltpu.VMEM((2, ROWS_PER_W), jnp.int32),
            pltpu.VMEM((2, 2, CH, PAD), jnp.float32),
            pltpu.SemaphoreType.DMA,
            pltpu.SemaphoreType.DMA,
            pltpu.SemaphoreType.DMA,
            pltpu.SemaphoreType.DMA,
        ],
    )
    def k(i0_hbm, i1_hbm, t0_hbm, t1_hbm, out0, out1,
          idx_v, rows_v, gsem0, gsem1, wsem0, wsem1):
        wid = lax.axis_index("s") * NUM_CORES + lax.axis_index("c")
        base = wid * ROWS_PER_W
        gsems = (gsem0, gsem1)
        wsems = (wsem0, wsem1)
        tables = (t0_hbm, t1_hbm)
        outs = (out0, out1)

        for t, src in enumerate((i0_hbm, i1_hbm)):
            pltpu.sync_copy(src.at[pl.ds(base, ROWS_PER_W)], idx_v.at[t])

        def fire_gathers(c, p):
            cps = []
            for t in range(2):
                cp = pltpu.make_async_copy(
                    tables[t].at[idx_v.at[t, pl.ds(c * CH, CH)]],
                    rows_v.at[p, t],
                    gsems[p])
                cp.start()
                cps.append(cp)
            return cps

        def fire_write(c, p):
            cps = []
            for t in range(2):
                cp = pltpu.make_async_copy(
                    rows_v.at[p, t],
                    outs[t].at[pl.ds(base + c * CH, CH)],
                    wsems[p])
                cp.start()
                cps.append(cp)
            return cps

        gathers = {0: fire_gathers(0, 0)}
        writes = {}
        for c in range(NCH):
            p, q = c % 2, (c + 1) % 2
            if c + 1 < NCH:
                if c >= 1:
                    for cp in writes.pop(c - 1):
                        cp.wait()
                gathers[c + 1] = fire_gathers(c + 1, q)
            for cp in gathers.pop(c):
                cp.wait()
            writes[c] = fire_write(c, p)
        for c in sorted(writes):
            for cp in writes.pop(c):
                cp.wait()

    return k(idx0, idx1, tab0, tab1)


def kernel(name_idx, head_idx, rel_idx, tail_idx, q_name, q_head, q_rel,
           names_w, heads_w, rels_w, tails_w, specials_w):
    name_all = jnp.concatenate([name_idx.astype(jnp.int32),
                                q_name.astype(jnp.int32)])
    head_all = jnp.concatenate([head_idx.astype(jnp.int32),
                                q_head.astype(jnp.int32)])
    rel_all = jnp.concatenate([rel_idx.astype(jnp.int32),
                               q_rel.astype(jnp.int32)])
    tail_all = jnp.concatenate([tail_idx.astype(jnp.int32),
                                jnp.zeros((1,), jnp.int32)])
    tails_p = jnp.pad(tails_w, ((0, 0), (0, PAD - EMB)))
    g_n, g_h = _sc_gather2(name_all, head_all,
                           _tc_relayout(names_w), _tc_relayout(heads_w))
    g_r, g_t = _sc_gather2(rel_all, tail_all,
                           _tc_relayout(rels_w), tails_p)
    g_t = g_t.at[NUM_ROWS - 1, :EMB].set(specials_w[MASK_ID])
    return _tc_assemble(g_n, g_h, g_r, g_t)


# speedup vs baseline: 1.2348x; 1.2348x over previous
import functools

import jax
import jax.numpy as jnp
from jax import lax
from jax.experimental import pallas as pl
from jax.experimental.pallas import tpu as pltpu
from jax.experimental.pallas import tpu_sc as plsc

NUM_ROWS = 16384
EMB = 64
NUM_COLS = 5 * EMB
PAD = 128
MASK_ID = 1
NUM_CORES = 2
NUM_SUBCORES = 16
NW = NUM_CORES * NUM_SUBCORES
ROWS_PER_W = NUM_ROWS // NW
CH = 128
NCH = ROWS_PER_W // CH


_TB = 8192


def _tc_relayout(table):
    v = table.shape[0]
    t_t = table.T

    def body(t_ref, out_ref):
        x = t_ref[...]
        xt = jnp.swapaxes(x, 0, 1)
        out_ref[...] = jnp.concatenate(
            [xt, jnp.zeros((_TB, EMB), jnp.float32)], axis=1)

    return pl.pallas_call(
        body,
        grid=(pl.cdiv(v, _TB),),
        in_specs=[pl.BlockSpec((EMB, _TB), lambda i: (0, i))],
        out_specs=pl.BlockSpec((_TB, PAD), lambda i: (i, 0)),
        out_shape=jax.ShapeDtypeStruct((v, PAD), jnp.float32),
    )(t_t)


def _tc_relayout2(tab_a, tab_b):
    v = tab_a.shape[0]
    a_t = tab_a.T
    b_t = tab_b.T

    def body(a_ref, b_ref, out_ref):
        out_ref[...] = jnp.concatenate(
            [jnp.swapaxes(a_ref[...], 0, 1),
             jnp.swapaxes(b_ref[...], 0, 1)], axis=1)

    spec = pl.BlockSpec((EMB, _TB), lambda i: (0, i))
    return pl.pallas_call(
        body,
        grid=(pl.cdiv(v, _TB),),
        in_specs=[spec, spec],
        out_specs=pl.BlockSpec((_TB, PAD), lambda i: (i, 0)),
        out_shape=jax.ShapeDtypeStruct((v, PAD), jnp.float32),
    )(a_t, b_t)


_AB = 2048


def _tc_assemble(g_n, g_h, g_r, g_t):
    def body(n_ref, h_ref, r_ref, t_ref, out_ref):
        n_t = jnp.swapaxes(n_ref[:, :EMB], 0, 1)
        h_t = jnp.swapaxes(h_ref[:, EMB:], 0, 1)
        r_t = jnp.swapaxes(r_ref[:, :EMB], 0, 1)
        t_t = jnp.swapaxes(t_ref[:, :EMB], 0, 1)
        out_ref[...] = jnp.concatenate([n_t, h_t, r_t, n_t, t_t], axis=0)

    spec = pl.BlockSpec((_AB, PAD), lambda j: (j, 0))
    out_t = pl.pallas_call(
        body,
        grid=(NUM_ROWS // _AB,),
        in_specs=[spec, spec, spec, spec],
        out_specs=pl.BlockSpec((NUM_COLS, _AB), lambda j: (0, j)),
        out_shape=jax.ShapeDtypeStruct((NUM_COLS, NUM_ROWS), jnp.float32),
    )(g_n, g_h, g_r, g_t)
    return out_t.T


def _sc_gather2(idx0, idx1, tab0, tab1):
    mesh = plsc.VectorSubcoreMesh(core_axis_name="c", subcore_axis_name="s")
    out_t = jax.ShapeDtypeStruct((NUM_ROWS, PAD), jnp.float32)

    @functools.partial(
        pl.kernel,
        mesh=mesh,
        compiler_params=pltpu.CompilerParams(use_tc_tiling_on_sc=False),
        out_type=(out_t, out_t),
        scratch_types=[
            pltpu.VMEM((2, ROWS_PER_W), jnp.int32),
            pltpu.VMEM((2, 2, CH, PAD), jnp.float32),
            pltpu.SemaphoreType.DMA,
            pltpu.SemaphoreType.DMA,
            pltpu.SemaphoreType.DMA,
            pltpu.SemaphoreType.DMA,
        ],
    )
    def k(i0_hbm, i1_hbm, t0_hbm, t1_hbm, out0, out1,
          idx_v, rows_v, gsem0, gsem1, wsem0, wsem1):
        wid = lax.axis_index("s") * NUM_CORES + lax.axis_index("c")
        base = wid * ROWS_PER_W
        gsems = (gsem0, gsem1)
        wsems = (wsem0, wsem1)
        tables = (t0_hbm, t1_hbm)
        outs = (out0, out1)

        for t, src in enumerate((i0_hbm, i1_hbm)):
            pltpu.sync_copy(src.at[pl.ds(base, ROWS_PER_W)], idx_v.at[t])

        def fire_gathers(c, p):
            cps = []
            for t in range(2):
                cp = pltpu.make_async_copy(
                    tables[t].at[idx_v.at[t, pl.ds(c * CH, CH)]],
                    rows_v.at[p, t],
                    gsems[p])
                cp.start()
                cps.append(cp)
            return cps

        def fire_write(c, p):
            cps = []
            for t in range(2):
                cp = pltpu.make_async_copy(
                    rows_v.at[p, t],
                    outs[t].at[pl.ds(base + c * CH, CH)],
                    wsems[p])
                cp.start()
                cps.append(cp)
            return cps

        gathers = {0: fire_gathers(0, 0)}
        writes = {}
        for c in range(NCH):
            p, q = c % 2, (c + 1) % 2
            if c + 1 < NCH:
                if c >= 1:
                    for cp in writes.pop(c - 1):
                        cp.wait()
                gathers[c + 1] = fire_gathers(c + 1, q)
            for cp in gathers.pop(c):
                cp.wait()
            writes[c] = fire_write(c, p)
        for c in sorted(writes):
            for cp in writes.pop(c):
                cp.wait()

    return k(idx0, idx1, tab0, tab1)


def kernel(name_idx, head_idx, rel_idx, tail_idx, q_name, q_head, q_rel,
           names_w, heads_w, rels_w, tails_w, specials_w):
    name_all = jnp.concatenate([name_idx.astype(jnp.int32),
                                q_name.astype(jnp.int32)])
    head_all = jnp.concatenate([head_idx.astype(jnp.int32),
                                q_head.astype(jnp.int32)])
    rel_all = jnp.concatenate([rel_idx.astype(jnp.int32),
                               q_rel.astype(jnp.int32)])
    tail_all = jnp.concatenate([tail_idx.astype(jnp.int32),
                                jnp.zeros((1,), jnp.int32)])
    nh_p = _tc_relayout2(names_w, heads_w)
    g_n, g_h = _sc_gather2(name_all, head_all, nh_p, nh_p)
    g_r, g_t = _sc_gather2(rel_all, tail_all,
                           _tc_relayout(rels_w), _tc_relayout(tails_w))
    g_t = g_t.at[NUM_ROWS - 1, :EMB].set(specials_w[MASK_ID])
    return _tc_assemble(g_n, g_h, g_r, g_t)
